# baseline (speedup 1.0000x reference)
import functools

import jax
import jax.numpy as jnp
from jax import lax
from jax.experimental import pallas as pl
from jax.experimental.pallas import tpu as pltpu
from jax.experimental.pallas import tpu_sc as plsc

DVEC = 64
SENT = 50
XP = 64
NC, NS, LANES = 2, 16, 16
NW = NC * NS


def _table_body(emb_ref, learn_ref, fixed_ref, cooc_ref, s_ref, x_ref):
    x_ref[...] = jnp.zeros_like(x_ref)
    x = jax.nn.softmax(emb_ref[...], axis=-1) * learn_ref[...] + fixed_ref[...]
    x_ref[pl.ds(0, SENT), :] = x
    xp = x_ref[...]
    for o in range(3):
        t = lax.dot_general(xp, cooc_ref[o], (((1,), (0,)), ((), ())),
                            preferred_element_type=jnp.float32)
        s_ref[o] = lax.dot_general(t, xp, (((1,), (1,)), ((), ())),
                                   preferred_element_type=jnp.float32)


def _score_table(emb_rows, learn_rows, fixed_rows, cooc):
    return pl.pallas_call(
        _table_body,
        in_specs=[
            pl.BlockSpec((SENT, DVEC), lambda: (0, 0)),
            pl.BlockSpec((SENT, 1), lambda: (0, 0)),
            pl.BlockSpec((SENT, DVEC), lambda: (0, 0)),
            pl.BlockSpec((3, DVEC, DVEC), lambda: (0, 0, 0)),
        ],
        out_specs=pl.BlockSpec((3, XP, XP), lambda: (0, 0, 0)),
        out_shape=jax.ShapeDtypeStruct((3, XP, XP), jnp.float32),
        scratch_shapes=[pltpu.VMEM((XP, DVEC), jnp.float32)],
    )(emb_rows, learn_rows[:, None], fixed_rows, cooc)


def _make_sc_kernel(k, n1):
    per_w = k // NW
    ch = per_w // LANES
    mesh = plsc.VectorSubcoreMesh(core_axis_name="c", subcore_axis_name="s")

    @functools.partial(
        pl.kernel,
        out_type=jax.ShapeDtypeStruct((k,), jnp.float32),
        mesh=mesh,
        compiler_params=pltpu.CompilerParams(needs_layout_passes=False),
        scratch_types=[
            pltpu.VMEM((3 * XP * XP,), jnp.float32),
            pltpu.VMEM((per_w * n1,), jnp.int32),
            pltpu.VMEM((per_w * n1,), jnp.int32),
            pltpu.VMEM((per_w * n1,), jnp.int32),
            pltpu.VMEM((SENT * LANES,), jnp.int32),
            pltpu.VMEM((per_w,), jnp.float32),
        ],
    )
    def sc_kernel(s_hbm, ops_hbm, f_hbm, a_hbm, out_hbm,
                  s_v, ops_v, f_v, a_v, ptr_v, sc_v):
        w = lax.axis_index("s") * NC + lax.axis_index("c")
        pltpu.sync_copy(s_hbm, s_v)
        pltpu.sync_copy(ops_hbm.at[pl.ds(w * per_w * n1, per_w * n1)], ops_v)
        pltpu.sync_copy(f_hbm.at[pl.ds(w * per_w * n1, per_w * n1)], f_v)
        pltpu.sync_copy(a_hbm.at[pl.ds(w * per_w * n1, per_w * n1)], a_v)
        lanes = lax.iota(jnp.int32, LANES)
        lanes_n1 = lanes * n1
        for j in range(ch):
            for s in range(SENT):
                plsc.store_scatter(ptr_v, [s * LANES + lanes],
                                   jnp.full((LANES,), s, jnp.int32))

            def step(i, acc, j=j):
                base = (j * LANES * n1 + i) + lanes_n1
                opv = plsc.load_gather(ops_v, [base])
                fv = plsc.load_gather(f_v, [base])
                av = plsc.load_gather(a_v, [base])
                pf = plsc.load_gather(ptr_v, [fv * LANES + lanes])
                pa = plsc.load_gather(ptr_v, [av * LANES + lanes])
                val = plsc.load_gather(s_v, [opv * (XP * XP) + pf * XP + pa])
                plsc.store_scatter(ptr_v, [fv * LANES + lanes], pa, mask=opv == 2)
                return acc + val

            acc = lax.fori_loop(0, n1, step, jnp.zeros((LANES,), jnp.float32))
            plsc.store_scatter(sc_v, [j * LANES + lanes], acc)
        pltpu.sync_copy(sc_v, out_hbm.at[pl.ds(w * per_w, per_w)])

    return sc_kernel


def kernel(emb_weight, learn_vectors, fixed_vectors, cooc, ids, ops, ix_func, ix_arg):
    onehot = (ids[:, None] == jnp.arange(emb_weight.shape[0])[None, :]).astype(jnp.float32)
    hi = jax.lax.Precision.HIGHEST
    emb_rows = jnp.matmul(onehot, emb_weight, precision=hi)
    fixed_rows = jnp.matmul(onehot, fixed_vectors, precision=hi)
    learn_rows = jnp.matmul(onehot, learn_vectors, precision=hi)
    s_pad = _score_table(emb_rows, learn_rows, fixed_rows, cooc)
    k, n1 = ops.shape
    sc_fn = _make_sc_kernel(k, n1)
    return sc_fn(s_pad.reshape(3 * XP * XP), ops.reshape(-1),
                 ix_func.reshape(-1), ix_arg.reshape(-1))


# speedup vs baseline: 1.1730x; 1.1730x over previous
import functools

import jax
import jax.numpy as jnp
from jax import lax
from jax.experimental import pallas as pl
from jax.experimental.pallas import tpu as pltpu
from jax.experimental.pallas import tpu_sc as plsc

DVEC = 64
SENT = 50
XP = 64
NC, NS, LANES = 2, 16, 16
NW = NC * NS


def _table_body(emb_ref, learn_ref, fixed_ref, cooc_ref, s_ref, x_ref):
    x_ref[...] = jnp.zeros_like(x_ref)
    x = jax.nn.softmax(emb_ref[...], axis=-1) * learn_ref[...] + fixed_ref[...]
    x_ref[pl.ds(0, SENT), :] = x
    xp = x_ref[...]
    for o in range(3):
        t = lax.dot_general(xp, cooc_ref[o], (((1,), (0,)), ((), ())),
                            preferred_element_type=jnp.float32)
        s_ref[o] = lax.dot_general(t, xp, (((1,), (1,)), ((), ())),
                                   preferred_element_type=jnp.float32)


def _score_table(emb_rows, learn_rows, fixed_rows, cooc):
    return pl.pallas_call(
        _table_body,
        in_specs=[
            pl.BlockSpec((SENT, DVEC), lambda: (0, 0)),
            pl.BlockSpec((SENT, 1), lambda: (0, 0)),
            pl.BlockSpec((SENT, DVEC), lambda: (0, 0)),
            pl.BlockSpec((3, DVEC, DVEC), lambda: (0, 0, 0)),
        ],
        out_specs=pl.BlockSpec((3, XP, XP), lambda: (0, 0, 0)),
        out_shape=jax.ShapeDtypeStruct((3, XP, XP), jnp.float32),
        scratch_shapes=[pltpu.VMEM((XP, DVEC), jnp.float32)],
    )(emb_rows, learn_rows[:, None], fixed_rows, cooc)


def _make_sc_kernel(k, n1):
    per_w = k // NW
    ch = per_w // LANES
    mesh = plsc.VectorSubcoreMesh(core_axis_name="c", subcore_axis_name="s")

    @functools.partial(
        pl.kernel,
        out_type=jax.ShapeDtypeStruct((k,), jnp.float32),
        mesh=mesh,
        compiler_params=pltpu.CompilerParams(needs_layout_passes=False),
        scratch_types=[
            pltpu.VMEM((3 * XP * XP,), jnp.float32),
            pltpu.VMEM((per_w * n1,), jnp.int32),
            pltpu.VMEM((per_w * n1,), jnp.int32),
            pltpu.VMEM((per_w * n1,), jnp.int32),
            pltpu.VMEM((SENT * LANES,), jnp.int32),
            pltpu.VMEM((per_w,), jnp.float32),
        ],
    )
    def sc_kernel(s_hbm, ops_hbm, f_hbm, a_hbm, out_hbm,
                  s_v, ops_v, f_v, a_v, ptr_v, sc_v):
        w = lax.axis_index("s") * NC + lax.axis_index("c")
        pltpu.sync_copy(s_hbm, s_v)
        pltpu.sync_copy(ops_hbm.at[pl.ds(w * per_w * n1, per_w * n1)], ops_v)
        pltpu.sync_copy(f_hbm.at[pl.ds(w * per_w * n1, per_w * n1)], f_v)
        pltpu.sync_copy(a_hbm.at[pl.ds(w * per_w * n1, per_w * n1)], a_v)
        lanes = lax.iota(jnp.int32, LANES)
        lanes_n1 = lanes * n1
        for j in range(ch):
            for s in range(SENT):
                plsc.store_scatter(ptr_v, [s * LANES + lanes],
                                   jnp.full((LANES,), s, jnp.int32))

            def step(i, acc, j=j):
                base = (j * LANES * n1 + i) + lanes_n1
                opv = plsc.load_gather(ops_v, [base])
                fv = plsc.load_gather(f_v, [base])
                av = plsc.load_gather(a_v, [base])
                pf = plsc.load_gather(ptr_v, [fv * LANES + lanes])
                pa = plsc.load_gather(ptr_v, [av * LANES + lanes])
                val = plsc.load_gather(s_v, [opv * (XP * XP) + pf * XP + pa])
                plsc.store_scatter(ptr_v, [fv * LANES + lanes], pa, mask=opv == 2)
                return acc + val

            acc = lax.fori_loop(0, n1, step, jnp.zeros((LANES,), jnp.float32))
            plsc.store_scatter(sc_v, [j * LANES + lanes], acc)
        pltpu.sync_copy(sc_v, out_hbm.at[pl.ds(w * per_w, per_w)])

    return sc_kernel


def kernel(emb_weight, learn_vectors, fixed_vectors, cooc, ids, ops, ix_func, ix_arg):
    onehot = (ids[:, None] == jnp.arange(emb_weight.shape[0])[None, :]).astype(jnp.float32)
    hi = jax.lax.Precision.HIGH
    emb_rows = jnp.matmul(onehot, emb_weight, precision=hi)
    fixed_rows = jnp.matmul(onehot, fixed_vectors, precision=hi)
    learn_rows = jnp.matmul(onehot, learn_vectors, precision=hi)
    s_pad = _score_table(emb_rows, learn_rows, fixed_rows, cooc)
    k, n1 = ops.shape
    sc_fn = _make_sc_kernel(k, n1)
    return sc_fn(s_pad.reshape(3 * XP * XP), ops.reshape(-1),
                 ix_func.reshape(-1), ix_arg.reshape(-1))


# speedup vs baseline: 1.4745x; 1.2570x over previous
import functools

import jax
import jax.numpy as jnp
from jax import lax
from jax.experimental import pallas as pl
from jax.experimental.pallas import tpu as pltpu
from jax.experimental.pallas import tpu_sc as plsc

DVEC = 64
SENT = 50
XP = 64
NC, NS, LANES = 2, 16, 16
NW = NC * NS


def _table_body(emb_ref, learn_ref, fixed_ref, cooc_ref, s_ref, x_ref):
    x_ref[...] = jnp.zeros_like(x_ref)
    x = jax.nn.softmax(emb_ref[...], axis=-1) * learn_ref[...] + fixed_ref[...]
    x_ref[pl.ds(0, SENT), :] = x
    xp = x_ref[pl.ds(0, XP), :]
    for o in range(3):
        t = lax.dot_general(xp, cooc_ref[o], (((1,), (0,)), ((), ())),
                            preferred_element_type=jnp.float32)
        s_ref[o] = lax.dot_general(t, x_ref[...], (((1,), (1,)), ((), ())),
                                   preferred_element_type=jnp.float32)


def _score_table(emb_rows, learn_rows, fixed_rows, cooc):
    return pl.pallas_call(
        _table_body,
        in_specs=[
            pl.BlockSpec((SENT, DVEC), lambda: (0, 0)),
            pl.BlockSpec((SENT, 1), lambda: (0, 0)),
            pl.BlockSpec((SENT, DVEC), lambda: (0, 0)),
            pl.BlockSpec((3, DVEC, DVEC), lambda: (0, 0, 0)),
        ],
        out_specs=pl.BlockSpec((3, XP, 128), lambda: (0, 0, 0)),
        out_shape=jax.ShapeDtypeStruct((3, XP, 128), jnp.float32),
        scratch_shapes=[pltpu.VMEM((128, DVEC), jnp.float32)],
    )(emb_rows, learn_rows[:, None], fixed_rows, cooc)


def _make_sc_kernel(k, n1):
    per_w = k // NW
    ch = per_w // LANES
    mesh = plsc.VectorSubcoreMesh(core_axis_name="c", subcore_axis_name="s")

    @functools.partial(
        pl.kernel,
        out_type=jax.ShapeDtypeStruct((k,), jnp.float32),
        mesh=mesh,
        compiler_params=pltpu.CompilerParams(needs_layout_passes=False),
        scratch_types=[
            pltpu.VMEM((3 * XP * 128,), jnp.float32),
            pltpu.VMEM((per_w * n1,), jnp.int32),
            pltpu.VMEM((SENT * LANES,), jnp.int32),
            pltpu.VMEM((per_w,), jnp.float32),
        ],
    )
    def sc_kernel(s_hbm, code_hbm, out_hbm, s_v, code_v, ptr_v, sc_v):
        w = lax.axis_index("s") * NC + lax.axis_index("c")
        pltpu.sync_copy(s_hbm, s_v)
        pltpu.sync_copy(code_hbm.at[pl.ds(w * per_w * n1, per_w * n1)], code_v)
        lanes = lax.iota(jnp.int32, LANES)
        lanes_n1 = lanes * n1
        for j in range(ch):
            for s in range(SENT):
                plsc.store_scatter(ptr_v, [s * LANES + lanes],
                                   jnp.full((LANES,), s, jnp.int32))

            def step(i, acc, j=j):
                code = plsc.load_gather(code_v, [(j * LANES * n1 + i) + lanes_n1])
                av = code & 63
                fv = (code >> 6) & 63
                opv = code >> 12
                pf = plsc.load_gather(ptr_v, [fv * LANES + lanes])
                pa = plsc.load_gather(ptr_v, [av * LANES + lanes])
                val = plsc.load_gather(s_v, [(opv << 13) + (pf << 7) + pa])
                plsc.store_scatter(ptr_v, [fv * LANES + lanes], pa, mask=opv == 2)
                return acc + val

            acc = lax.fori_loop(0, n1, step, jnp.zeros((LANES,), jnp.float32))
            plsc.store_scatter(sc_v, [j * LANES + lanes], acc)
        pltpu.sync_copy(sc_v, out_hbm.at[pl.ds(w * per_w, per_w)])

    return sc_kernel


def kernel(emb_weight, learn_vectors, fixed_vectors, cooc, ids, ops, ix_func, ix_arg):
    onehot = (ids[:, None] == jnp.arange(emb_weight.shape[0])[None, :]).astype(jnp.float32)
    hi = jax.lax.Precision.HIGH
    emb_rows = jnp.matmul(onehot, emb_weight, precision=hi)
    fixed_rows = jnp.matmul(onehot, fixed_vectors, precision=hi)
    learn_rows = jnp.take(learn_vectors, ids, axis=0)
    s_pad = _score_table(emb_rows, learn_rows, fixed_rows, cooc)
    k, n1 = ops.shape
    code = (ops << 12) | (ix_func << 6) | ix_arg
    sc_fn = _make_sc_kernel(k, n1)
    return sc_fn(s_pad.reshape(3 * XP * 128), code.reshape(-1))


# speedup vs baseline: 1.5199x; 1.0308x over previous
import functools

import jax
import jax.numpy as jnp
from jax import lax
from jax.experimental import pallas as pl
from jax.experimental.pallas import tpu as pltpu
from jax.experimental.pallas import tpu_sc as plsc

DVEC = 64
SENT = 50
XP = 64
NC, NS, LANES = 2, 16, 16
NW = NC * NS


def _table_body(emb_ref, learn_ref, fixed_ref, cooc_ref, s_ref, x_ref):
    x_ref[...] = jnp.zeros_like(x_ref)
    x = jax.nn.softmax(emb_ref[...], axis=-1) * learn_ref[...] + fixed_ref[...]
    x_ref[pl.ds(0, SENT), :] = x
    xp = x_ref[pl.ds(0, XP), :]
    for o in range(3):
        t = lax.dot_general(xp, cooc_ref[o], (((1,), (0,)), ((), ())),
                            preferred_element_type=jnp.float32)
        s_ref[o] = lax.dot_general(t, x_ref[...], (((1,), (1,)), ((), ())),
                                   preferred_element_type=jnp.float32)


def _score_table(emb_rows, learn_rows, fixed_rows, cooc):
    return pl.pallas_call(
        _table_body,
        in_specs=[
            pl.BlockSpec((SENT, DVEC), lambda: (0, 0)),
            pl.BlockSpec((SENT, 1), lambda: (0, 0)),
            pl.BlockSpec((SENT, DVEC), lambda: (0, 0)),
            pl.BlockSpec((3, DVEC, DVEC), lambda: (0, 0, 0)),
        ],
        out_specs=pl.BlockSpec((3, XP, 128), lambda: (0, 0, 0)),
        out_shape=jax.ShapeDtypeStruct((3, XP, 128), jnp.float32),
        scratch_shapes=[pltpu.VMEM((128, DVEC), jnp.float32)],
    )(emb_rows, learn_rows[:, None], fixed_rows, cooc)


def _make_sc_kernel(k, n1):
    per_w = k // NW
    ch = per_w // LANES
    mesh = plsc.VectorSubcoreMesh(core_axis_name="c", subcore_axis_name="s")

    @functools.partial(
        pl.kernel,
        out_type=jax.ShapeDtypeStruct((k,), jnp.float32),
        mesh=mesh,
        compiler_params=pltpu.CompilerParams(needs_layout_passes=False),
        scratch_types=[
            pltpu.VMEM((3 * XP * 128,), jnp.float32),
            pltpu.VMEM((per_w * n1,), jnp.int32),
            pltpu.VMEM((2 * SENT * LANES,), jnp.int32),
            pltpu.VMEM((per_w,), jnp.float32),
        ],
    )
    def sc_kernel(s_hbm, code_hbm, out_hbm, s_v, code_v, ptr_v, sc_v):
        w = lax.axis_index("s") * NC + lax.axis_index("c")
        pltpu.sync_copy(s_hbm, s_v)
        pltpu.sync_copy(code_hbm.at[pl.ds(w * per_w * n1, per_w * n1)], code_v)
        lanes = lax.iota(jnp.int32, LANES)
        lanes_n1 = lanes * n1
        sl = SENT * LANES
        for j in range(0, ch, 2):
            for s in range(SENT):
                splat = jnp.full((LANES,), s, jnp.int32)
                plsc.store_scatter(ptr_v, [s * LANES + lanes], splat)
                plsc.store_scatter(ptr_v, [sl + s * LANES + lanes], splat)

            def step(i, accs, j=j):
                a1, a2 = accs
                c1 = plsc.load_gather(code_v, [(j * LANES * n1 + i) + lanes_n1])
                c2 = plsc.load_gather(code_v, [((j + 1) * LANES * n1 + i) + lanes_n1])
                av1, av2 = c1 & 63, c2 & 63
                fv1, fv2 = (c1 >> 6) & 63, (c2 >> 6) & 63
                op1, op2 = c1 >> 12, c2 >> 12
                pf1 = plsc.load_gather(ptr_v, [fv1 * LANES + lanes])
                pf2 = plsc.load_gather(ptr_v, [sl + fv2 * LANES + lanes])
                pa1 = plsc.load_gather(ptr_v, [av1 * LANES + lanes])
                pa2 = plsc.load_gather(ptr_v, [sl + av2 * LANES + lanes])
                v1 = plsc.load_gather(s_v, [(op1 << 13) + (pf1 << 7) + pa1])
                v2 = plsc.load_gather(s_v, [(op2 << 13) + (pf2 << 7) + pa2])
                plsc.store_scatter(ptr_v, [fv1 * LANES + lanes], pa1, mask=op1 == 2)
                plsc.store_scatter(ptr_v, [sl + fv2 * LANES + lanes], pa2, mask=op2 == 2)
                return (a1 + v1, a2 + v2)

            zero = jnp.zeros((LANES,), jnp.float32)
            acc1, acc2 = lax.fori_loop(0, n1, step, (zero, zero))
            plsc.store_scatter(sc_v, [j * LANES + lanes], acc1)
            plsc.store_scatter(sc_v, [(j + 1) * LANES + lanes], acc2)
        pltpu.sync_copy(sc_v, out_hbm.at[pl.ds(w * per_w, per_w)])

    return sc_kernel


def kernel(emb_weight, learn_vectors, fixed_vectors, cooc, ids, ops, ix_func, ix_arg):
    onehot = (ids[:, None] == jnp.arange(emb_weight.shape[0])[None, :]).astype(jnp.float32)
    hi = jax.lax.Precision.HIGH
    emb_rows = jnp.matmul(onehot, emb_weight, precision=hi)
    fixed_rows = jnp.matmul(onehot, fixed_vectors, precision=hi)
    learn_rows = jnp.take(learn_vectors, ids, axis=0)
    s_pad = _score_table(emb_rows, learn_rows, fixed_rows, cooc)
    k, n1 = ops.shape
    code = (ops << 12) | (ix_func << 6) | ix_arg
    sc_fn = _make_sc_kernel(k, n1)
    return sc_fn(s_pad.reshape(3 * XP * 128), code.reshape(-1))
